# baseline (device time: 187671 ns/iter reference)
import jax
import jax.numpy as jnp
from jax import lax
from jax.experimental import pallas as pl
from jax.experimental.pallas import tpu as pltpu

N_DEV = 4
SQ = 2048
SKV = 2048
D_MODEL = 1024
H_PER = 8
DH = 128
CHUNK = SQ // N_DEV
HALF = CHUNK // 2
SCALE = 0.08838834764831843


def _fused_body(x_ref, wq_ref, k_ref, v_ref, wo_ref, out_ref,
                bias_ref, comm_ref, send_sems, recv_sems, credit_sem,
                sr_send, sr_recv, sl_send, sl_recv):
    my = lax.axis_index("i")
    right = (my + 1) % N_DEV
    left = (my - 1) % N_DEV

    def compute_chunk(c):
        rows = pl.ds(c * CHUNK, CHUNK)
        row = c * CHUNK + lax.broadcasted_iota(jnp.int32, (CHUNK, SKV), 0)
        col = lax.broadcasted_iota(jnp.int32, (CHUNK, SKV), 1)
        qb = row // 64
        kb = col // 64
        mask = (qb == kb) | (kb == 0) | (((qb + kb) % 3) == 0)
        bias_ref[...] = jnp.where(mask, 0.0, -1e9)

        xb = x_ref[rows, :]
        out_ref[rows, :] = jnp.zeros((CHUNK, D_MODEL), jnp.float32)

        def h_body(h, _):
            q = jnp.dot(xb, wq_ref[h], preferred_element_type=jnp.float32)
            k = k_ref[:, pl.ds(h * DH, DH)]
            v = v_ref[:, pl.ds(h * DH, DH)]
            s = lax.dot_general(
                q, k, (((1,), (1,)), ((), ())),
                preferred_element_type=jnp.float32,
            ) + bias_ref[...]
            w = jnp.exp(s)
            denom = jnp.sum(w, axis=1, keepdims=True)
            ctx = jnp.dot(w, v, preferred_element_type=jnp.float32)
            ctx = ctx / denom
            out_ref[rows, :] += jnp.dot(ctx, wo_ref[h],
                                        preferred_element_type=jnp.float32)
            return _

        lax.fori_loop(0, H_PER, h_body, None)

    sends = []

    c0 = (my + 3) % N_DEV
    compute_chunk(c0)
    send0 = pltpu.make_async_remote_copy(
        src_ref=out_ref.at[pl.ds(c0 * CHUNK, CHUNK), :],
        dst_ref=comm_ref.at[0],
        send_sem=send_sems.at[0],
        recv_sem=recv_sems.at[0],
        device_id=(right,),
        device_id_type=pl.DeviceIdType.MESH,
    )
    send0.start()
    sends.append(send0)

    for s in range(1, N_DEV):
        c = (my + 3 - s) % N_DEV
        compute_chunk(c)
        rows = pl.ds(c * CHUNK, CHUNK)
        slot = (s - 1) % 2
        recv = pltpu.make_async_remote_copy(
            src_ref=comm_ref.at[slot],
            dst_ref=comm_ref.at[slot],
            send_sem=send_sems.at[slot],
            recv_sem=recv_sems.at[slot],
            device_id=(left,),
            device_id_type=pl.DeviceIdType.MESH,
        )
        recv.wait_recv()
        out_ref[rows, :] += comm_ref[slot]
        if s == 1:
            pl.semaphore_signal(
                credit_sem, inc=1,
                device_id=(left,), device_id_type=pl.DeviceIdType.MESH,
            )
        if s < N_DEV - 1:
            if s == 2:
                pl.semaphore_wait(credit_sem, 1)
                send0.wait_send()
            snd = pltpu.make_async_remote_copy(
                src_ref=out_ref.at[rows, :],
                dst_ref=comm_ref.at[s % 2],
                send_sem=send_sems.at[s % 2],
                recv_sem=recv_sems.at[s % 2],
                device_id=(right,),
                device_id_type=pl.DeviceIdType.MESH,
            )
            snd.start()
            sends.append(snd)

    for t in range(N_DEV - 1):
        cr = (my - t) % N_DEV
        cl = (my + t) % N_DEV
        slot = t % 2
        ra = pltpu.make_async_remote_copy(
            src_ref=out_ref.at[pl.ds(cr * CHUNK, HALF), :],
            dst_ref=out_ref.at[pl.ds(cr * CHUNK, HALF), :],
            send_sem=sr_send.at[slot],
            recv_sem=sr_recv.at[slot],
            device_id=(right,),
            device_id_type=pl.DeviceIdType.MESH,
        )
        rb = pltpu.make_async_remote_copy(
            src_ref=out_ref.at[pl.ds(cl * CHUNK + HALF, HALF), :],
            dst_ref=out_ref.at[pl.ds(cl * CHUNK + HALF, HALF), :],
            send_sem=sl_send.at[slot],
            recv_sem=sl_recv.at[slot],
            device_id=(left,),
            device_id_type=pl.DeviceIdType.MESH,
        )
        ra.start()
        rb.start()
        ra.wait()
        rb.wait()

    sends[1].wait_send()
    sends[2].wait_send()


def kernel(x, Wq, K_ext, V_ext, Wo):
    my = lax.axis_index("i")
    x2d = x.reshape(SQ, D_MODEL)
    Wq_loc = lax.dynamic_slice(Wq, (0, my * (H_PER * DH)), (D_MODEL, H_PER * DH))
    Wo_loc = lax.dynamic_slice(Wo, (my * (H_PER * DH), 0), (H_PER * DH, D_MODEL))
    Wq_h = Wq_loc.reshape(D_MODEL, H_PER, DH).transpose(1, 0, 2) * SCALE
    Wo_h = Wo_loc.reshape(H_PER, DH, D_MODEL)
    K = K_ext.reshape(SKV, H_PER * DH)
    V = V_ext.reshape(SKV, H_PER * DH)

    out = pl.pallas_call(
        _fused_body,
        out_shape=jax.ShapeDtypeStruct((SQ, D_MODEL), jnp.float32),
        in_specs=[pl.BlockSpec(memory_space=pltpu.VMEM)] * 5,
        out_specs=pl.BlockSpec(memory_space=pltpu.VMEM),
        scratch_shapes=[
            pltpu.VMEM((CHUNK, SKV), jnp.float32),
            pltpu.VMEM((2, CHUNK, D_MODEL), jnp.float32),
            pltpu.SemaphoreType.DMA((2,)),
            pltpu.SemaphoreType.DMA((2,)),
            pltpu.SemaphoreType.REGULAR,
            pltpu.SemaphoreType.DMA((2,)),
            pltpu.SemaphoreType.DMA((2,)),
            pltpu.SemaphoreType.DMA((2,)),
            pltpu.SemaphoreType.DMA((2,)),
        ],
        compiler_params=pltpu.CompilerParams(
            vmem_limit_bytes=100 * 1024 * 1024,
        ),
    )(x2d, Wq_h, K, V, Wo_h)
    return out.reshape(1, SQ, D_MODEL)


# device time: 186017 ns/iter; 1.0089x vs baseline; 1.0089x over previous
import jax
import jax.numpy as jnp
from jax import lax
from jax.experimental import pallas as pl
from jax.experimental.pallas import tpu as pltpu

N_DEV = 4
SQ = 2048
SKV = 2048
D_MODEL = 1024
H_PER = 8
DH = 128
CHUNK = SQ // N_DEV
HALF = CHUNK // 2
SCALE = 0.08838834764831843


def _fused_body(x_ref, wq_ref, k_ref, v_ref, wo_ref, out_ref,
                bias_ref, comm_ref, send_sems, recv_sems, credit_sem,
                sr_send, sr_recv, sl_send, sl_recv):
    my = lax.axis_index("i")
    right = (my + 1) % N_DEV
    left = (my - 1) % N_DEV

    def compute_chunk(c):
        rows = pl.ds(c * CHUNK, CHUNK)
        row = c * CHUNK + lax.broadcasted_iota(jnp.int32, (CHUNK, SKV), 0)
        col = lax.broadcasted_iota(jnp.int32, (CHUNK, SKV), 1)
        qb = row // 64
        kb = col // 64
        mask = (qb == kb) | (kb == 0) | (((qb + kb) % 3) == 0)
        bias_ref[...] = jnp.where(mask, 0.0, -1e9)

        xb = x_ref[rows, :]
        out_ref[rows, :] = jnp.zeros((CHUNK, D_MODEL), jnp.float32)

        def h_body(h, _):
            q = jnp.dot(xb, wq_ref[h], preferred_element_type=jnp.float32)
            s = lax.dot_general(
                q, k_ref[h], (((1,), (1,)), ((), ())),
                preferred_element_type=jnp.float32,
            ) + bias_ref[...]
            w = jnp.exp(s)
            denom = jnp.sum(w, axis=1, keepdims=True)
            ctx = jnp.dot(w, v_ref[h], preferred_element_type=jnp.float32)
            ctx = ctx / denom
            out_ref[rows, :] += jnp.dot(ctx, wo_ref[h],
                                        preferred_element_type=jnp.float32)
            return _

        lax.fori_loop(0, H_PER, h_body, None)

    sends = []

    c0 = (my + 3) % N_DEV
    compute_chunk(c0)
    send0 = pltpu.make_async_remote_copy(
        src_ref=out_ref.at[pl.ds(c0 * CHUNK, CHUNK), :],
        dst_ref=comm_ref.at[0],
        send_sem=send_sems.at[0],
        recv_sem=recv_sems.at[0],
        device_id=(right,),
        device_id_type=pl.DeviceIdType.MESH,
    )
    send0.start()
    sends.append(send0)

    for s in range(1, N_DEV):
        c = (my + 3 - s) % N_DEV
        compute_chunk(c)
        rows = pl.ds(c * CHUNK, CHUNK)
        slot = (s - 1) % 2
        recv = pltpu.make_async_remote_copy(
            src_ref=comm_ref.at[slot],
            dst_ref=comm_ref.at[slot],
            send_sem=send_sems.at[slot],
            recv_sem=recv_sems.at[slot],
            device_id=(left,),
            device_id_type=pl.DeviceIdType.MESH,
        )
        recv.wait_recv()
        out_ref[rows, :] += comm_ref[slot]
        if s == 1:
            pl.semaphore_signal(
                credit_sem, inc=1,
                device_id=(left,), device_id_type=pl.DeviceIdType.MESH,
            )
        if s < N_DEV - 1:
            if s == 2:
                pl.semaphore_wait(credit_sem, 1)
                send0.wait_send()
            snd = pltpu.make_async_remote_copy(
                src_ref=out_ref.at[rows, :],
                dst_ref=comm_ref.at[s % 2],
                send_sem=send_sems.at[s % 2],
                recv_sem=recv_sems.at[s % 2],
                device_id=(right,),
                device_id_type=pl.DeviceIdType.MESH,
            )
            snd.start()
            sends.append(snd)

    for t in range(N_DEV - 1):
        cr = (my - t) % N_DEV
        cl = (my + t) % N_DEV
        slot = t % 2
        ra = pltpu.make_async_remote_copy(
            src_ref=out_ref.at[pl.ds(cr * CHUNK, HALF), :],
            dst_ref=out_ref.at[pl.ds(cr * CHUNK, HALF), :],
            send_sem=sr_send.at[slot],
            recv_sem=sr_recv.at[slot],
            device_id=(right,),
            device_id_type=pl.DeviceIdType.MESH,
        )
        rb = pltpu.make_async_remote_copy(
            src_ref=out_ref.at[pl.ds(cl * CHUNK + HALF, HALF), :],
            dst_ref=out_ref.at[pl.ds(cl * CHUNK + HALF, HALF), :],
            send_sem=sl_send.at[slot],
            recv_sem=sl_recv.at[slot],
            device_id=(left,),
            device_id_type=pl.DeviceIdType.MESH,
        )
        ra.start()
        rb.start()
        ra.wait()
        rb.wait()

    sends[1].wait_send()
    sends[2].wait_send()


def kernel(x, Wq, K_ext, V_ext, Wo):
    my = lax.axis_index("i")
    x2d = x.reshape(SQ, D_MODEL)
    Wq_loc = lax.dynamic_slice(Wq, (0, my * (H_PER * DH)), (D_MODEL, H_PER * DH))
    Wo_loc = lax.dynamic_slice(Wo, (my * (H_PER * DH), 0), (H_PER * DH, D_MODEL))
    Wq_h = Wq_loc.reshape(D_MODEL, H_PER, DH).transpose(1, 0, 2) * SCALE
    Wo_h = Wo_loc.reshape(H_PER, DH, D_MODEL)
    K = K_ext.reshape(SKV, H_PER, DH).transpose(1, 0, 2)
    V = V_ext.reshape(SKV, H_PER, DH).transpose(1, 0, 2)

    out = pl.pallas_call(
        _fused_body,
        out_shape=jax.ShapeDtypeStruct((SQ, D_MODEL), jnp.float32),
        in_specs=[pl.BlockSpec(memory_space=pltpu.VMEM)] * 5,
        out_specs=pl.BlockSpec(memory_space=pltpu.VMEM),
        scratch_shapes=[
            pltpu.VMEM((CHUNK, SKV), jnp.float32),
            pltpu.VMEM((2, CHUNK, D_MODEL), jnp.float32),
            pltpu.SemaphoreType.DMA((2,)),
            pltpu.SemaphoreType.DMA((2,)),
            pltpu.SemaphoreType.REGULAR,
            pltpu.SemaphoreType.DMA((2,)),
            pltpu.SemaphoreType.DMA((2,)),
            pltpu.SemaphoreType.DMA((2,)),
            pltpu.SemaphoreType.DMA((2,)),
        ],
        compiler_params=pltpu.CompilerParams(
            vmem_limit_bytes=100 * 1024 * 1024,
        ),
    )(x2d, Wq_h, K, V, Wo_h)
    return out.reshape(1, SQ, D_MODEL)
